# baseline (speedup 1.0000x reference)
import functools

import jax
import jax.numpy as jnp
from jax import lax
from jax.experimental import pallas as pl
from jax.experimental.pallas import tpu as pltpu
from jax.experimental.pallas import tpu_sc as plsc

HIDDEN = 2048
NCOL = 8
CW = HIDDEN // NCOL
NUM_CB = 32
CB_VOCAB = 2051
NC, NS, L = 2, 16, 16
NW = NC * NS
GP = 16
RING = 16
UNROLL = 2


def _emb_call(n_pos, audio_tok, text_ids, text_table, audio_table):
    ppw = n_pos // NW
    ngrp = NCOL * (ppw // GP)
    nunit = ngrp * 2 * GP
    mesh = plsc.VectorSubcoreMesh(core_axis_name="c", subcore_axis_name="s")

    @functools.partial(
        pl.kernel,
        out_type=jax.ShapeDtypeStruct((n_pos, HIDDEN), jnp.float32),
        mesh=mesh,
        scratch_types=[
            pltpu.VMEM((2, GP, NUM_CB), jnp.int32),
            pltpu.VMEM((ppw,), jnp.int32),
            pltpu.VMEM((RING, L, CW), jnp.float32),
            pltpu.VMEM((2, GP, CW), jnp.float32),
        ] + [pltpu.SemaphoreType.DMA] * 19,
    )
    def k(atok_hbm, tids_hbm, text_hbm, audio_hbm, out_hbm,
          atok_v, tids_v, bufs, out_v,
          sem_t, *rest_sems):
        wid = lax.axis_index("s") * NC + lax.axis_index("c")
        lane = lax.iota(jnp.int32, 16)
        base_pos = wid * ppw
        pltpu.sync_copy(atok_hbm.at[pl.ds(base_pos, GP)], atok_v.at[0])
        pltpu.sync_copy(tids_hbm.at[pl.ds(base_pos, ppw)], tids_v)
        gsems = rest_sems[:16]
        osems = rest_sems[16:18]

        def fire(u, i):
            g = u >> 5
            col = g & (NCOL - 1)
            bk = (g >> 3) & 1
            fp = (u >> 1) & (GP - 1)
            sh = u & 1
            v = atok_v[bk, fp, pl.ds(sh * L, L)]
            ix = jnp.where(v == 0, 0, v + (lane + sh * L) * CB_VOCAB)
            return pltpu.async_copy(
                audio_hbm.at[ix, pl.ds(col * CW, CW)], bufs.at[i], gsems[i])

        def fire_text(g):
            o = g & 1
            col = g & (NCOL - 1)
            tix = tids_v[pl.ds((g >> 3) * GP, GP)]
            return pltpu.async_copy(
                text_hbm.at[tix, pl.ds(col * CW, CW)], out_v.at[o], sem_t)

        def acc(u, i):
            o = (u >> 5) & 1
            row = (u >> 1) & (GP - 1)

            @plsc.parallel_loop(0, CW // L, unroll=UNROLL)
            def _(c):
                off = c * L
                s = bufs[i, 0, pl.ds(off, L)]
                for j in range(1, L):
                    s = s + bufs[i, j, pl.ds(off, L)]
                plsc.addupdate(out_v.at[o, row, pl.ds(off, L)], s)

        def out_dst(g):
            col = g & (NCOL - 1)
            return out_hbm.at[pl.ds(base_pos + (g >> 3) * GP, GP),
                              pl.ds(col * CW, CW)]

        def store(g, parity):
            return pltpu.async_copy(out_v.at[parity], out_dst(g),
                                    osems[parity])

        def drain_store(g, parity):
            pltpu.make_async_copy(out_v.at[parity], out_dst(g),
                                  osems[parity]).wait()

        fire_text(0).wait()
        for i in range(RING):
            fire(i, i)

        def it_body(it, _):
            for kk in range(16):
                u = it * 16 + kk
                if kk == 0:
                    g = u >> 5

                    @pl.when(jnp.logical_and(it % 2 == 0, it > 0))
                    def _():
                        @pl.when(jnp.logical_and(g >= 2, g % 2 == 0))
                        def _():
                            drain_store(g - 2, 0)

                        @pl.when(jnp.logical_and(g >= 2, g % 2 == 1))
                        def _():
                            drain_store(g - 2, 1)

                        @pl.when(jnp.logical_and(g & (NCOL - 1) == NCOL - 1,
                                                 g < ngrp - 1))
                        def _():
                            blk1 = (g >> 3) + 1
                            pltpu.sync_copy(
                                atok_hbm.at[pl.ds(base_pos + blk1 * GP, GP)],
                                atok_v.at[blk1 & 1])
                        fire_text(g).wait()
                pltpu.make_async_copy(
                    audio_hbm.at[pl.ds(0, L), pl.ds(0, CW)],
                    bufs.at[kk], gsems[kk]).wait()
                acc(u, kk)

                @pl.when(u + RING < nunit)
                def _():
                    fire(u + RING, kk)
                if kk == 15:
                    g7 = u >> 5

                    @pl.when(jnp.logical_and(it % 2 == 1, g7 % 2 == 0))
                    def _():
                        store(g7, 0)

                    @pl.when(jnp.logical_and(it % 2 == 1, g7 % 2 == 1))
                    def _():
                        store(g7, 1)
            return 0

        lax.fori_loop(0, nunit // 16, it_body, 0)
        drain_store(ngrp - 2, (ngrp - 2) & 1)
        drain_store(ngrp - 1, (ngrp - 1) & 1)

    return k(audio_tok, text_ids, text_table, audio_table)


def kernel(input_ids, text_table, audio_table, audio_tokens_offsets):
    b, s, _ = input_ids.shape
    n_pos = b * s
    ids2 = input_ids.reshape(n_pos, NUM_CB + 1).astype(jnp.int32)
    audio_tok = ids2[:, :NUM_CB]
    text_ids = ids2[:, NUM_CB]
    out = _emb_call(n_pos, audio_tok, text_ids, text_table, audio_table)
    return out.reshape(b, s, HIDDEN)


# speedup vs baseline: 1.5070x; 1.5070x over previous
import functools

import jax
import jax.numpy as jnp
from jax import lax
from jax.experimental import pallas as pl
from jax.experimental.pallas import tpu as pltpu
from jax.experimental.pallas import tpu_sc as plsc

HIDDEN = 2048
NCOL = 4
CW = HIDDEN // NCOL
NUM_CB = 32
CB_VOCAB = 2051
NC, NS, L = 2, 16, 16
NW = NC * NS
GP = 16
RING = 8
UNROLL = 2


def _emb_call(n_pos, audio_tok, text_ids, text_table, audio_table):
    ppw = n_pos // NW
    ngrp = NCOL * (ppw // GP)
    nunit = ngrp * 2 * GP
    mesh = plsc.VectorSubcoreMesh(core_axis_name="c", subcore_axis_name="s")

    @functools.partial(
        pl.kernel,
        out_type=jax.ShapeDtypeStruct((n_pos, HIDDEN), jnp.float32),
        mesh=mesh,
        scratch_types=[
            pltpu.VMEM((2, GP, NUM_CB), jnp.int32),
            pltpu.VMEM((ppw,), jnp.int32),
            pltpu.VMEM((RING, L, CW), jnp.float32),
            pltpu.VMEM((2, GP, CW), jnp.float32),
            pltpu.SemaphoreType.DMA,
            pltpu.SemaphoreType.DMA,
            pltpu.SemaphoreType.DMA,
            pltpu.SemaphoreType.DMA,
            pltpu.SemaphoreType.DMA,
            pltpu.SemaphoreType.DMA,
            pltpu.SemaphoreType.DMA,
            pltpu.SemaphoreType.DMA,
            pltpu.SemaphoreType.DMA,
            pltpu.SemaphoreType.DMA,
            pltpu.SemaphoreType.DMA,
        ],
    )
    def k(atok_hbm, tids_hbm, text_hbm, audio_hbm, out_hbm,
          atok_v, tids_v, bufs, out_v,
          sem_t, sem_g0, sem_g1, sem_g2, sem_g3, sem_g4, sem_g5, sem_g6,
          sem_g7, sem_o0, sem_o1):
        wid = lax.axis_index("s") * NC + lax.axis_index("c")
        lane = lax.iota(jnp.int32, 16)
        base_pos = wid * ppw
        pltpu.sync_copy(atok_hbm.at[pl.ds(base_pos, GP)], atok_v.at[0])
        pltpu.sync_copy(tids_hbm.at[pl.ds(base_pos, ppw)], tids_v)
        gsems = (sem_g0, sem_g1, sem_g2, sem_g3, sem_g4, sem_g5, sem_g6,
                 sem_g7)
        osems = (sem_o0, sem_o1)

        def fire(u, i):
            g = u >> 5
            col = g & (NCOL - 1)
            bk = (g >> 2) & 1
            fp = (u >> 1) & (GP - 1)
            sh = u & 1
            v = atok_v[bk, fp, pl.ds(sh * L, L)]
            ix = jnp.where(v == 0, 0, v + (lane + sh * L) * CB_VOCAB)
            return pltpu.async_copy(
                audio_hbm.at[ix, pl.ds(col * CW, CW)], bufs.at[i], gsems[i])

        def fire_text(g):
            o = g & 1
            col = g & (NCOL - 1)
            tix = tids_v[pl.ds((g >> 2) * GP, GP)]
            return pltpu.async_copy(
                text_hbm.at[tix, pl.ds(col * CW, CW)], out_v.at[o], sem_t)

        def acc(u, i):
            o = (u >> 5) & 1
            row = (u >> 1) & (GP - 1)

            @plsc.parallel_loop(0, CW // L, unroll=UNROLL)
            def _(c):
                off = c * L
                s = bufs[i, 0, pl.ds(off, L)]
                for j in range(1, L):
                    s = s + bufs[i, j, pl.ds(off, L)]
                plsc.addupdate(out_v.at[o, row, pl.ds(off, L)], s)

        def out_dst(g):
            col = g & (NCOL - 1)
            return out_hbm.at[pl.ds(base_pos + (g >> 2) * GP, GP),
                              pl.ds(col * CW, CW)]

        def store(g, parity):
            return pltpu.async_copy(out_v.at[parity], out_dst(g),
                                    osems[parity])

        def drain_store(g, parity):
            pltpu.make_async_copy(out_v.at[parity], out_dst(g),
                                  osems[parity]).wait()

        fire_text(0).wait()
        for i in range(RING):
            fire(i, i)

        def it_body(it, _):
            for kk in range(8):
                u = it * 8 + kk
                if kk == 0:
                    g = u >> 5

                    @pl.when(jnp.logical_and(it % 4 == 0, it > 0))
                    def _():
                        @pl.when(jnp.logical_and(g >= 2, g % 2 == 0))
                        def _():
                            drain_store(g - 2, 0)

                        @pl.when(jnp.logical_and(g >= 2, g % 2 == 1))
                        def _():
                            drain_store(g - 2, 1)

                        @pl.when(jnp.logical_and(g & (NCOL - 1) == NCOL - 1,
                                                 g < ngrp - 1))
                        def _():
                            blk1 = (g >> 2) + 1
                            pltpu.sync_copy(
                                atok_hbm.at[pl.ds(base_pos + blk1 * GP, GP)],
                                atok_v.at[blk1 & 1])
                        fire_text(g).wait()
                pltpu.make_async_copy(
                    audio_hbm.at[pl.ds(0, L), pl.ds(0, CW)],
                    bufs.at[kk], gsems[kk]).wait()
                acc(u, kk)

                @pl.when(u + RING < nunit)
                def _():
                    fire(u + RING, kk)
                if kk == 7:
                    g7 = u >> 5

                    @pl.when(jnp.logical_and(it % 4 == 3, g7 % 2 == 0))
                    def _():
                        store(g7, 0)

                    @pl.when(jnp.logical_and(it % 4 == 3, g7 % 2 == 1))
                    def _():
                        store(g7, 1)
            return 0

        lax.fori_loop(0, nunit // 8, it_body, 0)
        drain_store(ngrp - 2, (ngrp - 2) & 1)
        drain_store(ngrp - 1, (ngrp - 1) & 1)

    return k(audio_tok, text_ids, text_table, audio_table)


def kernel(input_ids, text_table, audio_table, audio_tokens_offsets):
    b, s, _ = input_ids.shape
    n_pos = b * s
    ids2 = input_ids.reshape(n_pos, NUM_CB + 1).astype(jnp.int32)
    audio_tok = ids2[:, :NUM_CB]
    text_ids = ids2[:, NUM_CB]
    out = _emb_call(n_pos, audio_tok, text_ids, text_table, audio_table)
    return out.reshape(b, s, HIDDEN)
